# Initial kernel scaffold; baseline (speedup 1.0000x reference)
import jax
import jax.numpy as jnp
from jax.experimental import pallas as pl


def kernel(x, edge_index, W, M):
    raise NotImplementedError("write your pallas kernel here")


# speedup vs baseline: 3.0400x; 3.0400x over previous
import functools

import jax
import jax.numpy as jnp
from jax import lax
from jax.experimental import pallas as pl
from jax.experimental.pallas import tpu as pltpu
from jax.experimental.pallas import tpu_sc as plsc

N_NODES = 10000
N_EDGES = 320000
D = 128

NC = 2
NS = 16
NW = NC * NS

N_PAD = 10240
ROWS_PER_TILE = N_PAD // NS
CHUNK = 128
EPT = 10240
E_PAD = EPT * NW
STEPS = EPT // CHUNK
WB_CHUNKS = ROWS_PER_TILE // CHUNK


def _sc_body(x_hbm, src_hbm, dst_hbm, out_hbm, idx_s, idx_d, rows, agg, sem):
    c = lax.axis_index("c")
    s = lax.axis_index("s")
    w = c * NS + s

    pltpu.sync_copy(src_hbm.at[pl.ds(w * STEPS, STEPS)], idx_s)
    pltpu.sync_copy(dst_hbm.at[pl.ds(w * STEPS, STEPS)], idx_d)

    z = jnp.zeros((16,), jnp.float32)

    def _zero_row(i, _):
        for k in range(8):
            rows[i, pl.ds(k * 16, 16)] = z
        return 0

    lax.fori_loop(0, CHUNK, _zero_row, 0)
    rbase = s * ROWS_PER_TILE
    for k in range(WB_CHUNKS):
        pltpu.sync_copy(rows, agg.at[pl.ds(rbase + k * CHUNK, CHUNK)])
    plsc.subcore_barrier()

    def _step(j, _):
        pltpu.async_copy(x_hbm.at[idx_s.at[j]], rows, sem).wait()
        pltpu.sync_copy(rows, agg.at[idx_d.at[j]], add=True)
        return 0

    lax.fori_loop(0, STEPS, _step, 0)
    plsc.subcore_barrier()

    for k in range(WB_CHUNKS):
        r0 = rbase + k * CHUNK
        pltpu.sync_copy(agg.at[pl.ds(r0, CHUNK)], rows)
        pltpu.sync_copy(rows, out_hbm.at[c, pl.ds(r0, CHUNK)])


_sc_agg = functools.partial(
    pl.kernel,
    out_type=jax.ShapeDtypeStruct((NC, N_PAD, D), jnp.float32),
    mesh=plsc.VectorSubcoreMesh(core_axis_name="c", subcore_axis_name="s"),
    scratch_types=[
        pltpu.VMEM((STEPS, CHUNK), jnp.int32),
        pltpu.VMEM((STEPS, CHUNK), jnp.int32),
        pltpu.VMEM((CHUNK, D), jnp.float32),
        pltpu.VMEM_SHARED((N_PAD, D), jnp.float32),
        pltpu.SemaphoreType.DMA,
    ],
)(_sc_body)


def _tc_body(x_ref, p_ref, w_ref, m_ref, o_ref):
    self_term = jnp.dot(x_ref[...], w_ref[...], preferred_element_type=jnp.float32)
    agg = p_ref[0] + p_ref[1]
    neigh = jnp.dot(agg, m_ref[...], preferred_element_type=jnp.float32)
    o_ref[...] = jnp.maximum(self_term + neigh, 0.0)


_TC_BLK = 1000


def _tc_combine(x, partials, W, M):
    return pl.pallas_call(
        _tc_body,
        grid=(N_NODES // _TC_BLK,),
        in_specs=[
            pl.BlockSpec((_TC_BLK, D), lambda i: (i, 0)),
            pl.BlockSpec((NC, _TC_BLK, D), lambda i: (0, i, 0)),
            pl.BlockSpec((D, D), lambda i: (0, 0)),
            pl.BlockSpec((D, D), lambda i: (0, 0)),
        ],
        out_specs=pl.BlockSpec((_TC_BLK, D), lambda i: (i, 0)),
        out_shape=jax.ShapeDtypeStruct((N_NODES, D), jnp.float32),
    )(x, partials, W, M)


@jax.jit
def kernel(x, edge_index, W, M):
    src = edge_index[0].astype(jnp.int32)
    dst = edge_index[1].astype(jnp.int32)
    pad = E_PAD - N_EDGES
    src_p = jnp.concatenate([src, jnp.zeros((pad,), jnp.int32)])
    dst_p = jnp.concatenate([dst, jnp.full((pad,), N_PAD - 1, jnp.int32)])
    src2 = src_p.reshape(-1, CHUNK)
    dst2 = dst_p.reshape(-1, CHUNK)
    partials = _sc_agg(x, src2, dst2)
    return _tc_combine(x, partials, W, M)


# speedup vs baseline: 3.2815x; 1.0795x over previous
import functools

import jax
import jax.numpy as jnp
from jax import lax
from jax.experimental import pallas as pl
from jax.experimental.pallas import tpu as pltpu
from jax.experimental.pallas import tpu_sc as plsc

N_NODES = 10000
N_EDGES = 320000
D = 128

NC = 2
NS = 16
NW = NC * NS

N_PAD = 10240
ROWS_PER_TILE = N_PAD // NS
CHUNK = 128
EPT = 10240
E_PAD = EPT * NW
STEPS = EPT // CHUNK
WB_CHUNKS = ROWS_PER_TILE // CHUNK


_DST_SHIFT = 14


def _sc_body(x_hbm, packed_hbm, out_hbm, idx_p, srcb, dstb, rows, agg, gsem):
    c = lax.axis_index("c")
    s = lax.axis_index("s")
    w = c * NS + s

    pltpu.sync_copy(packed_hbm.at[pl.ds(w * STEPS, STEPS)], idx_p)

    z = jnp.zeros((16,), jnp.float32)

    def _zero_row(i, _):
        for k in range(8):
            rows[0, i, pl.ds(k * 16, 16)] = z
        return 0

    lax.fori_loop(0, CHUNK, _zero_row, 0)
    rbase = s * ROWS_PER_TILE
    for k in range(WB_CHUNKS):
        pltpu.sync_copy(rows.at[0], agg.at[pl.ds(rbase + k * CHUNK, CHUNK)])
    plsc.subcore_barrier()

    def _unpack(j, pb):
        for k in range(CHUNK // 16):
            v = idx_p[j, pl.ds(k * 16, 16)]
            srcb[pb, pl.ds(k * 16, 16)] = v & ((1 << _DST_SHIFT) - 1)
            dstb[pb, pl.ds(k * 16, 16)] = lax.shift_right_logical(v, _DST_SHIFT)

    _unpack(0, 0)
    pltpu.async_copy(x_hbm.at[srcb.at[0]], rows.at[0], gsem)

    def _outer(t, _):
        for p in range(2):
            j = t * 2 + p
            pltpu.make_async_copy(x_hbm.at[srcb.at[p]], rows.at[p], gsem).wait()

            @pl.when(j + 1 < STEPS)
            def _():
                _unpack(j + 1, 1 - p)
                pltpu.async_copy(
                    x_hbm.at[srcb.at[1 - p]], rows.at[1 - p], gsem
                )

            pltpu.sync_copy(rows.at[p], agg.at[dstb.at[p]], add=True)
        return 0

    lax.fori_loop(0, STEPS // 2, _outer, 0)
    plsc.subcore_barrier()

    for k in range(WB_CHUNKS):
        r0 = rbase + k * CHUNK
        b = k % 2
        pltpu.sync_copy(agg.at[pl.ds(r0, CHUNK)], rows.at[b])
        pltpu.sync_copy(rows.at[b], out_hbm.at[c, pl.ds(r0, CHUNK)])


_sc_agg = functools.partial(
    pl.kernel,
    out_type=jax.ShapeDtypeStruct((NC, N_PAD, D), jnp.float32),
    mesh=plsc.VectorSubcoreMesh(core_axis_name="c", subcore_axis_name="s"),
    scratch_types=[
        pltpu.VMEM((STEPS, CHUNK), jnp.int32),
        pltpu.VMEM((2, CHUNK), jnp.int32),
        pltpu.VMEM((2, CHUNK), jnp.int32),
        pltpu.VMEM((2, CHUNK, D), jnp.float32),
        pltpu.VMEM_SHARED((N_PAD, D), jnp.float32),
        pltpu.SemaphoreType.DMA,
    ],
)(_sc_body)


def _tc_body(x_ref, p_ref, w_ref, m_ref, o_ref):
    self_term = jnp.dot(x_ref[...], w_ref[...], preferred_element_type=jnp.float32)
    agg = p_ref[0] + p_ref[1]
    neigh = jnp.dot(agg, m_ref[...], preferred_element_type=jnp.float32)
    o_ref[...] = jnp.maximum(self_term + neigh, 0.0)


_TC_BLK = 1000


def _tc_combine(x, partials, W, M):
    return pl.pallas_call(
        _tc_body,
        grid=(N_NODES // _TC_BLK,),
        in_specs=[
            pl.BlockSpec((_TC_BLK, D), lambda i: (i, 0)),
            pl.BlockSpec((NC, _TC_BLK, D), lambda i: (0, i, 0)),
            pl.BlockSpec((D, D), lambda i: (0, 0)),
            pl.BlockSpec((D, D), lambda i: (0, 0)),
        ],
        out_specs=pl.BlockSpec((_TC_BLK, D), lambda i: (i, 0)),
        out_shape=jax.ShapeDtypeStruct((N_NODES, D), jnp.float32),
    )(x, partials, W, M)


@jax.jit
def kernel(x, edge_index, W, M):
    src = edge_index[0].astype(jnp.int32)
    dst = edge_index[1].astype(jnp.int32)
    pad = E_PAD - N_EDGES
    packed = src | (dst << _DST_SHIFT)
    pad_val = jnp.int32((N_PAD - 1) << _DST_SHIFT)
    packed = jnp.concatenate([packed, jnp.full((pad,), pad_val, jnp.int32)])
    partials = _sc_agg(x, packed.reshape(-1, CHUNK))
    return _tc_combine(x, partials, W, M)


# speedup vs baseline: 3.3849x; 1.0315x over previous
import functools

import jax
import jax.numpy as jnp
from jax import lax
from jax.experimental import pallas as pl
from jax.experimental.pallas import tpu as pltpu
from jax.experimental.pallas import tpu_sc as plsc

N_NODES = 10000
N_EDGES = 320000
D = 128

NC = 2
NS = 16
NW = NC * NS

N_PAD = 10240
ROWS_PER_TILE = N_PAD // NS
LANES = 128
CHUNK = 64
NB = 4
EPT = 10240
E_PAD = EPT * NW
IDX_ROWS = EPT // LANES
STEPS = EPT // CHUNK
WB_CHUNKS = ROWS_PER_TILE // CHUNK


_DST_SHIFT = 14


def _sc_body(x_hbm, packed_hbm, out_hbm, idx_p, srcb, dstb, rows, agg, gsem):
    c = lax.axis_index("c")
    s = lax.axis_index("s")
    w = c * NS + s

    pltpu.sync_copy(packed_hbm.at[pl.ds(w * IDX_ROWS, IDX_ROWS)], idx_p)

    z = jnp.zeros((16,), jnp.float32)

    def _zero_row(i, _):
        for k in range(8):
            rows[0, i, pl.ds(k * 16, 16)] = z
        return 0

    lax.fori_loop(0, CHUNK, _zero_row, 0)
    rbase = s * ROWS_PER_TILE
    for k in range(WB_CHUNKS):
        pltpu.sync_copy(rows.at[0], agg.at[pl.ds(rbase + k * CHUNK, CHUNK)])
    plsc.subcore_barrier()

    def _unpack(j, pb):
        row = lax.shift_right_logical(j, 1)
        off = (j & 1) * CHUNK
        for k in range(CHUNK // 16):
            v = idx_p[row, pl.ds(off + k * 16, 16)]
            srcb[pb, pl.ds(k * 16, 16)] = v & ((1 << _DST_SHIFT) - 1)
            dstb[pb, pl.ds(k * 16, 16)] = lax.shift_right_logical(v, _DST_SHIFT)

    for b in range(NB - 1):
        _unpack(jnp.int32(b), b)
        pltpu.async_copy(x_hbm.at[srcb.at[b]], rows.at[b], gsem)

    def _outer(t, _):
        for b in range(NB):
            j = t * NB + b
            pltpu.make_async_copy(x_hbm.at[srcb.at[b]], rows.at[b], gsem).wait()
            nxt = (b + NB - 1) % NB

            @pl.when(j + NB - 1 < STEPS)
            def _():
                _unpack(j + NB - 1, nxt)
                pltpu.async_copy(x_hbm.at[srcb.at[nxt]], rows.at[nxt], gsem)

            pltpu.sync_copy(rows.at[b], agg.at[dstb.at[b]], add=True)
        return 0

    lax.fori_loop(0, STEPS // NB, _outer, 0)
    plsc.subcore_barrier()

    for k in range(WB_CHUNKS):
        r0 = rbase + k * CHUNK
        b = k % NB
        pltpu.sync_copy(agg.at[pl.ds(r0, CHUNK)], rows.at[b])
        pltpu.sync_copy(rows.at[b], out_hbm.at[c, pl.ds(r0, CHUNK)])


_sc_agg = functools.partial(
    pl.kernel,
    out_type=jax.ShapeDtypeStruct((NC, N_PAD, D), jnp.float32),
    mesh=plsc.VectorSubcoreMesh(core_axis_name="c", subcore_axis_name="s"),
    scratch_types=[
        pltpu.VMEM((IDX_ROWS, LANES), jnp.int32),
        pltpu.VMEM((NB, CHUNK), jnp.int32),
        pltpu.VMEM((NB, CHUNK), jnp.int32),
        pltpu.VMEM((NB, CHUNK, D), jnp.float32),
        pltpu.VMEM_SHARED((N_PAD, D), jnp.float32),
        pltpu.SemaphoreType.DMA,
    ],
)(_sc_body)


def _tc_body(x_ref, p_ref, w_ref, m_ref, o_ref):
    self_term = jnp.dot(x_ref[...], w_ref[...], preferred_element_type=jnp.float32)
    agg = p_ref[0] + p_ref[1]
    neigh = jnp.dot(agg, m_ref[...], preferred_element_type=jnp.float32)
    o_ref[...] = jnp.maximum(self_term + neigh, 0.0)


_TC_BLK = 1000


def _tc_combine(x, partials, W, M):
    return pl.pallas_call(
        _tc_body,
        grid=(N_NODES // _TC_BLK,),
        in_specs=[
            pl.BlockSpec((_TC_BLK, D), lambda i: (i, 0)),
            pl.BlockSpec((NC, _TC_BLK, D), lambda i: (0, i, 0)),
            pl.BlockSpec((D, D), lambda i: (0, 0)),
            pl.BlockSpec((D, D), lambda i: (0, 0)),
        ],
        out_specs=pl.BlockSpec((_TC_BLK, D), lambda i: (i, 0)),
        out_shape=jax.ShapeDtypeStruct((N_NODES, D), jnp.float32),
    )(x, partials, W, M)


@jax.jit
def kernel(x, edge_index, W, M):
    src = edge_index[0].astype(jnp.int32)
    dst = edge_index[1].astype(jnp.int32)
    pad = E_PAD - N_EDGES
    packed = src | (dst << _DST_SHIFT)
    pad_val = jnp.int32((N_PAD - 1) << _DST_SHIFT)
    packed = jnp.concatenate([packed, jnp.full((pad,), pad_val, jnp.int32)])
    partials = _sc_agg(x, packed.reshape(-1, LANES))
    return _tc_combine(x, partials, W, M)


# speedup vs baseline: 3.3854x; 1.0002x over previous
import functools

import jax
import jax.numpy as jnp
from jax import lax
from jax.experimental import pallas as pl
from jax.experimental.pallas import tpu as pltpu
from jax.experimental.pallas import tpu_sc as plsc

N_NODES = 10000
N_EDGES = 320000
D = 128

NC = 2
NS = 16
NW = NC * NS

N_PAD = 10240
ROWS_PER_TILE = N_PAD // NS
LANES = 128
CHUNK = 64
NB = 4
EPT = 10240
E_PAD = EPT * NW
IDX_ROWS = EPT // LANES
STEPS = EPT // CHUNK
WB_CHUNKS = ROWS_PER_TILE // CHUNK


_DST_SHIFT = 14


def _sc_body(x_hbm, packed_hbm, out_hbm, idx_p, srcb, dstb, rows, agg, *gsems):
    c = lax.axis_index("c")
    s = lax.axis_index("s")
    w = c * NS + s

    pltpu.sync_copy(packed_hbm.at[pl.ds(w * IDX_ROWS, IDX_ROWS)], idx_p)

    z = jnp.zeros((16,), jnp.float32)

    def _zero_row(i, _):
        for k in range(8):
            rows[0, i, pl.ds(k * 16, 16)] = z
        return 0

    lax.fori_loop(0, CHUNK, _zero_row, 0)
    rbase = s * ROWS_PER_TILE
    for k in range(WB_CHUNKS):
        pltpu.sync_copy(rows.at[0], agg.at[pl.ds(rbase + k * CHUNK, CHUNK)])
    plsc.subcore_barrier()

    def _unpack(j, pb):
        row = lax.shift_right_logical(j, 1)
        off = (j & 1) * CHUNK
        for k in range(CHUNK // 16):
            v = idx_p[row, pl.ds(off + k * 16, 16)]
            srcb[pb, pl.ds(k * 16, 16)] = v & ((1 << _DST_SHIFT) - 1)
            dstb[pb, pl.ds(k * 16, 16)] = lax.shift_right_logical(v, _DST_SHIFT)

    for b in range(NB - 1):
        _unpack(jnp.int32(b), b)
        pltpu.async_copy(x_hbm.at[srcb.at[b]], rows.at[b], gsems[b])

    def _outer(t, _):
        for b in range(NB):
            j = t * NB + b
            pltpu.make_async_copy(x_hbm.at[srcb.at[b]], rows.at[b], gsems[b]).wait()
            nxt = (b + NB - 1) % NB

            @pl.when(j + NB - 1 < STEPS)
            def _():
                _unpack(j + NB - 1, nxt)
                pltpu.async_copy(x_hbm.at[srcb.at[nxt]], rows.at[nxt], gsems[nxt])

            pltpu.sync_copy(rows.at[b], agg.at[dstb.at[b]], add=True)
        return 0

    lax.fori_loop(0, STEPS // NB, _outer, 0)
    plsc.subcore_barrier()

    for k in range(WB_CHUNKS):
        r0 = rbase + k * CHUNK
        b = k % NB
        pltpu.sync_copy(agg.at[pl.ds(r0, CHUNK)], rows.at[b])
        pltpu.sync_copy(rows.at[b], out_hbm.at[c, pl.ds(r0, CHUNK)])


_sc_agg = functools.partial(
    pl.kernel,
    out_type=jax.ShapeDtypeStruct((NC, N_PAD, D), jnp.float32),
    mesh=plsc.VectorSubcoreMesh(core_axis_name="c", subcore_axis_name="s"),
    scratch_types=[
        pltpu.VMEM((IDX_ROWS, LANES), jnp.int32),
        pltpu.VMEM((NB, CHUNK), jnp.int32),
        pltpu.VMEM((NB, CHUNK), jnp.int32),
        pltpu.VMEM((NB, CHUNK, D), jnp.float32),
        pltpu.VMEM_SHARED((N_PAD, D), jnp.float32),
        pltpu.SemaphoreType.DMA,
        pltpu.SemaphoreType.DMA,
        pltpu.SemaphoreType.DMA,
        pltpu.SemaphoreType.DMA,
    ],
)(_sc_body)


def _tc_body(x_ref, p_ref, w_ref, m_ref, o_ref):
    self_term = jnp.dot(x_ref[...], w_ref[...], preferred_element_type=jnp.float32)
    agg = p_ref[0] + p_ref[1]
    neigh = jnp.dot(agg, m_ref[...], preferred_element_type=jnp.float32)
    o_ref[...] = jnp.maximum(self_term + neigh, 0.0)


_TC_BLK = 1000


def _tc_combine(x, partials, W, M):
    return pl.pallas_call(
        _tc_body,
        grid=(N_NODES // _TC_BLK,),
        in_specs=[
            pl.BlockSpec((_TC_BLK, D), lambda i: (i, 0)),
            pl.BlockSpec((NC, _TC_BLK, D), lambda i: (0, i, 0)),
            pl.BlockSpec((D, D), lambda i: (0, 0)),
            pl.BlockSpec((D, D), lambda i: (0, 0)),
        ],
        out_specs=pl.BlockSpec((_TC_BLK, D), lambda i: (i, 0)),
        out_shape=jax.ShapeDtypeStruct((N_NODES, D), jnp.float32),
    )(x, partials, W, M)


@jax.jit
def kernel(x, edge_index, W, M):
    src = edge_index[0].astype(jnp.int32)
    dst = edge_index[1].astype(jnp.int32)
    pad = E_PAD - N_EDGES
    packed = src | (dst << _DST_SHIFT)
    pad_val = jnp.int32((N_PAD - 1) << _DST_SHIFT)
    packed = jnp.concatenate([packed, jnp.full((pad,), pad_val, jnp.int32)])
    partials = _sc_agg(x, packed.reshape(-1, LANES))
    return _tc_combine(x, partials, W, M)
